# Initial kernel scaffold; baseline (speedup 1.0000x reference)
import jax
import jax.numpy as jnp
from jax.experimental import pallas as pl


def kernel(layer0, layer1, layer2):
    raise NotImplementedError("write your pallas kernel here")


# speedup vs baseline: 48.3238x; 48.3238x over previous
import functools

import numpy as np
import jax
import jax.numpy as jnp
from jax import lax
from jax.experimental import pallas as pl
from jax.experimental.pallas import tpu as pltpu

_BINS = 6
_WIDTH = np.float32(256.0 / _BINS)
_OUT = 240
_BIG = np.float32(3.0e38)


def _resize_matrix(src):
    x = (np.arange(_OUT, dtype=np.float64) + 0.5) * (src / _OUT) - 0.5
    lo = np.floor(x).astype(np.int64)
    frac = x - lo
    a = np.zeros((_OUT, src), np.float64)
    for i in range(_OUT):
        for tap, wt in ((lo[i], 1.0 - frac[i]), (lo[i] + 1, frac[i])):
            a[i, min(max(int(tap), 0), src - 1)] += wt
    return a.astype(np.float32)


def _fmin6(vals):
    return functools.reduce(jnp.minimum, vals)


def _fmax6(vals):
    return functools.reduce(jnp.maximum, vals)


def _layer_body(x_ref, a_ref, at_ref, col_ref, acc_ref):
    c = pl.program_id(0)
    nprog = pl.num_programs(0)
    t = x_ref[0]
    h, w = t.shape
    rows = lax.broadcasted_iota(jnp.int32, (h, w), 0)
    cols = lax.broadcasted_iota(jnp.int32, (h, w), 1)
    border = (rows == 0) | (rows == h - 1) | (cols == 0) | (cols == w - 1)
    t = jnp.where(border, jnp.float32(0.0), t)

    tmin = jnp.min(t)
    tmax = jnp.max(t)
    rng = tmax - tmin
    deg = rng == 0.0
    safe = jnp.where(deg, jnp.float32(1.0), rng)
    ch = jnp.where(deg, 0.0, (t - tmin) / safe * 256.0)
    chb = jnp.where(deg, 0.0, (0.0 - tmin) / safe * 256.0)

    bin1 = jnp.clip(jnp.floor(ch / _WIDTH), 0.0, 5.0)
    hidx = jnp.clip(jnp.floor(ch * 6.0 - 1.0), 0.0, 5.0)
    hb = jnp.clip(jnp.floor(chb * 6.0 - 1.0), 0.0, 5.0)

    n = jnp.float32(h * w)
    nb = jnp.float32(2 * h + 2 * w - 4)

    c1 = [jnp.sum(jnp.where(bin1 == b, 1.0, 0.0)) for b in range(_BINS)]
    c2 = [jnp.sum(jnp.where(hidx == b, 1.0, 0.0)) for b in range(_BINS)]

    lv = [-jnp.log(c1[b] / n + 1e-4) for b in range(_BINS)]
    pres = [c2[b] > 0.0 for b in range(_BINS)]
    dmin = _fmin6([jnp.where(pres[b], lv[b], _BIG) for b in range(_BINS)])
    dmax = _fmax6([jnp.where(pres[b], lv[b], -_BIG) for b in range(_BINS)])
    drng = dmax - dmin
    ddeg = drng == 0.0
    dsafe = jnp.where(ddeg, 1.0, drng)
    ln = [jnp.where(ddeg, 0.0, (lv[b] - dmin) / dsafe) for b in range(_BINS)]
    lmax = _fmax6([jnp.where(pres[b], ln[b], -_BIG) for b in range(_BINS)])
    lmean = sum(c2[b] * ln[b] for b in range(_BINS)) / n
    w_r = (lmax - lmean) ** 2
    rv = [ln[b] * w_r for b in range(_BINS)]

    rminp = _fmin6([jnp.where(pres[b], rv[b], _BIG) for b in range(_BINS)])
    rmaxp = _fmax6([jnp.where(pres[b], rv[b], -_BIG) for b in range(_BINS)])
    rmean = sum(c2[b] * rv[b] for b in range(_BINS)) / n
    w0 = (rmaxp - rmean) ** 2
    frng = rmaxp - rminp
    fdeg = frng == 0.0
    fsafe = jnp.where(fdeg, 1.0, frng)
    t0 = [jnp.where(fdeg, 0.0, (rv[b] - rminp) / fsafe * w0)
          for b in range(_BINS)]
    sb0 = sum(jnp.where(hb == b, t0[b], 0.0) for b in range(_BINS))

    cint = [c2[b] - nb * jnp.where(hb == b, 1.0, 0.0) for b in range(_BINS)]
    presi = [cint[b] > 0.0 for b in range(_BINS)]
    zmin = jnp.minimum(
        0.0, _fmin6([jnp.where(presi[b], rv[b], _BIG) for b in range(_BINS)]))
    zmax = jnp.maximum(
        0.0, _fmax6([jnp.where(presi[b], rv[b], -_BIG) for b in range(_BINS)]))
    zmean = sum(cint[b] * rv[b] for b in range(_BINS)) / n
    wz = (zmax - zmean) ** 2
    zrng = zmax - zmin
    zdeg = zrng == 0.0
    zsafe = jnp.where(zdeg, 1.0, zrng)
    tz = [jnp.where(zdeg, 0.0, (rv[b] - zmin) / zsafe * wz)
          for b in range(_BINS)]
    bz = jnp.where(zdeg, 0.0, (0.0 - zmin) / zsafe * wz)

    is0 = c == 0
    tab = [jnp.where(is0, t0[b], tz[b]) for b in range(_BINS)]
    sb = jnp.where(is0, sb0, bz)

    g = tab[5]
    for b in (4, 3, 2, 1, 0):
        g = jnp.where(hidx == b, tab[b], g)
    contrib = jnp.where(border, sb, g)

    @pl.when(is0)
    def _init():
        acc_ref[...] = jnp.zeros_like(acc_ref)

    acc_ref[...] += contrib

    @pl.when(c == nprog - 1)
    def _finish():
        p = acc_ref[...]
        pmin = jnp.min(p)
        pmax = jnp.max(p)
        prng = pmax - pmin
        pdeg = prng == 0.0
        psafe = jnp.where(pdeg, 1.0, prng)
        pn = jnp.where(pdeg, 0.0, (p - pmin) / psafe)
        pt = jnp.where(pn < 0.2, 0.0, pn)
        tmp = jnp.dot(a_ref[...], pt, preferred_element_type=jnp.float32)
        r = jnp.dot(tmp, at_ref[...], preferred_element_type=jnp.float32)
        rmin = jnp.min(r)
        rmax = jnp.max(r)
        rrng = rmax - rmin
        rdeg = rrng == 0.0
        rsafe = jnp.where(rdeg, 1.0, rrng)
        col_ref[...] = jnp.where(rdeg, 0.0, (r - rmin) / rsafe * 256.0)


def _layer_col(x, amat, atmat):
    cdim, h, w = x.shape
    return pl.pallas_call(
        _layer_body,
        grid=(cdim,),
        in_specs=[
            pl.BlockSpec((1, h, w), lambda c: (c, 0, 0)),
            pl.BlockSpec((_OUT, h), lambda c: (0, 0)),
            pl.BlockSpec((h, _OUT), lambda c: (0, 0)),
        ],
        out_specs=pl.BlockSpec((_OUT, _OUT), lambda c: (0, 0)),
        out_shape=jax.ShapeDtypeStruct((_OUT, _OUT), jnp.float32),
        scratch_shapes=[pltpu.VMEM((h, w), jnp.float32)],
    )(x, amat, atmat)


_A112 = _resize_matrix(112)
_A56 = _resize_matrix(56)
_A28 = _resize_matrix(28)


def kernel(layer0, layer1, layer2):
    cols = []
    for x, a in ((layer0, _A112), (layer1, _A56), (layer2, _A28)):
        cols.append(_layer_col(
            x[0], jnp.asarray(a), jnp.asarray(np.ascontiguousarray(a.T))))
    groups = jnp.stack(cols, axis=-1)
    return groups.sum(axis=-1), groups


# speedup vs baseline: 229.0529x; 4.7400x over previous
import functools

import numpy as np
import jax
import jax.numpy as jnp
from jax import lax
from jax.experimental import pallas as pl
from jax.experimental.pallas import tpu as pltpu

_BINS = 6
_WIDTH = np.float32(256.0 / _BINS)
_OUT = 240
_BIG = np.float32(3.0e38)


def _resize_matrix(src):
    x = (np.arange(_OUT, dtype=np.float64) + 0.5) * (src / _OUT) - 0.5
    lo = np.floor(x).astype(np.int64)
    frac = x - lo
    a = np.zeros((_OUT, src), np.float64)
    for i in range(_OUT):
        for tap, wt in ((lo[i], 1.0 - frac[i]), (lo[i] + 1, frac[i])):
            a[i, min(max(int(tap), 0), src - 1)] += wt
    return a.astype(np.float32)


def _fmin6(vals):
    return functools.reduce(jnp.minimum, vals)


def _fmax6(vals):
    return functools.reduce(jnp.maximum, vals)


def _rsum(x):
    return jnp.sum(x, axis=(1, 2), keepdims=True)


def _layer_body(x_ref, a_ref, at_ref, col_ref):
    t = x_ref[...]
    cdim, h, w = t.shape
    rows = lax.broadcasted_iota(jnp.int32, (h, w), 0)
    cols = lax.broadcasted_iota(jnp.int32, (h, w), 1)
    border = ((rows == 0) | (rows == h - 1) | (cols == 0) | (cols == w - 1))
    border = border[None]
    t = jnp.where(border, jnp.float32(0.0), t)

    tmin = jnp.min(t, axis=(1, 2), keepdims=True)
    tmax = jnp.max(t, axis=(1, 2), keepdims=True)
    rng = tmax - tmin
    deg = rng == 0.0
    safe = jnp.where(deg, jnp.float32(1.0), rng)
    ch = jnp.where(deg, 0.0, (t - tmin) / safe * 256.0)
    chb = jnp.where(deg, 0.0, (0.0 - tmin) / safe * 256.0)

    bin1 = jnp.clip(jnp.floor(ch / _WIDTH), 0.0, 5.0)
    hidx = jnp.clip(jnp.floor(ch * 6.0 - 1.0), 0.0, 5.0)
    hb = jnp.clip(jnp.floor(chb * 6.0 - 1.0), 0.0, 5.0)

    n = jnp.float32(h * w)
    nb = jnp.float32(2 * h + 2 * w - 4)

    c1 = [_rsum(jnp.where(bin1 == b, 1.0, 0.0)) for b in range(_BINS)]
    c2 = [_rsum(jnp.where(hidx == b, 1.0, 0.0)) for b in range(_BINS)]

    lv = [-jnp.log(c1[b] / n + 1e-4) for b in range(_BINS)]
    pres = [c2[b] > 0.0 for b in range(_BINS)]
    dmin = _fmin6([jnp.where(pres[b], lv[b], _BIG) for b in range(_BINS)])
    dmax = _fmax6([jnp.where(pres[b], lv[b], -_BIG) for b in range(_BINS)])
    drng = dmax - dmin
    ddeg = drng == 0.0
    dsafe = jnp.where(ddeg, 1.0, drng)
    ln = [jnp.where(ddeg, 0.0, (lv[b] - dmin) / dsafe) for b in range(_BINS)]
    lmax = _fmax6([jnp.where(pres[b], ln[b], -_BIG) for b in range(_BINS)])
    lmean = sum(c2[b] * ln[b] for b in range(_BINS)) / n
    w_r = (lmax - lmean) ** 2
    rv = [ln[b] * w_r for b in range(_BINS)]

    rminp = _fmin6([jnp.where(pres[b], rv[b], _BIG) for b in range(_BINS)])
    rmaxp = _fmax6([jnp.where(pres[b], rv[b], -_BIG) for b in range(_BINS)])
    rmean = sum(c2[b] * rv[b] for b in range(_BINS)) / n
    w0 = (rmaxp - rmean) ** 2
    frng = rmaxp - rminp
    fdeg = frng == 0.0
    fsafe = jnp.where(fdeg, 1.0, frng)
    t0 = [jnp.where(fdeg, 0.0, (rv[b] - rminp) / fsafe * w0)
          for b in range(_BINS)]
    sb0 = sum(jnp.where(hb == b, t0[b], 0.0) for b in range(_BINS))

    cint = [c2[b] - nb * jnp.where(hb == b, 1.0, 0.0) for b in range(_BINS)]
    presi = [cint[b] > 0.0 for b in range(_BINS)]
    zmin = jnp.minimum(
        0.0, _fmin6([jnp.where(presi[b], rv[b], _BIG) for b in range(_BINS)]))
    zmax = jnp.maximum(
        0.0, _fmax6([jnp.where(presi[b], rv[b], -_BIG) for b in range(_BINS)]))
    zmean = sum(cint[b] * rv[b] for b in range(_BINS)) / n
    wz = (zmax - zmean) ** 2
    zrng = zmax - zmin
    zdeg = zrng == 0.0
    zsafe = jnp.where(zdeg, 1.0, zrng)
    tz = [jnp.where(zdeg, 0.0, (rv[b] - zmin) / zsafe * wz)
          for b in range(_BINS)]
    bz = jnp.where(zdeg, 0.0, (0.0 - zmin) / zsafe * wz)

    is0 = lax.broadcasted_iota(jnp.int32, (cdim, 1, 1), 0) == 0
    tab = [jnp.where(is0, t0[b], tz[b]) for b in range(_BINS)]
    sb = jnp.where(is0, sb0, bz)

    g = tab[5]
    for b in (4, 3, 2, 1, 0):
        g = jnp.where(hidx == b, tab[b], g)
    contrib = jnp.where(border, sb, g)

    p = jnp.sum(contrib, axis=0)

    pmin = jnp.min(p)
    pmax = jnp.max(p)
    prng = pmax - pmin
    pdeg = prng == 0.0
    psafe = jnp.where(pdeg, 1.0, prng)
    pn = jnp.where(pdeg, 0.0, (p - pmin) / psafe)
    pt = jnp.where(pn < 0.2, 0.0, pn)
    tmp = jnp.dot(a_ref[...], pt, preferred_element_type=jnp.float32)
    r = jnp.dot(tmp, at_ref[...], preferred_element_type=jnp.float32)
    rmin = jnp.min(r)
    rmax = jnp.max(r)
    rrng = rmax - rmin
    rdeg = rrng == 0.0
    rsafe = jnp.where(rdeg, 1.0, rrng)
    col_ref[...] = jnp.where(rdeg, 0.0, (r - rmin) / rsafe * 256.0)


def _layer_col(x, amat, atmat):
    cdim, h, w = x.shape
    return pl.pallas_call(
        _layer_body,
        grid=(1,),
        in_specs=[
            pl.BlockSpec((cdim, h, w), lambda i: (0, 0, 0)),
            pl.BlockSpec((_OUT, h), lambda i: (0, 0)),
            pl.BlockSpec((h, _OUT), lambda i: (0, 0)),
        ],
        out_specs=pl.BlockSpec((_OUT, _OUT), lambda i: (0, 0)),
        out_shape=jax.ShapeDtypeStruct((_OUT, _OUT), jnp.float32),
    )(x, amat, atmat)


_A112 = _resize_matrix(112)
_A56 = _resize_matrix(56)
_A28 = _resize_matrix(28)


def kernel(layer0, layer1, layer2):
    cols = []
    for x, a in ((layer0, _A112), (layer1, _A56), (layer2, _A28)):
        cols.append(_layer_col(
            x[0], jnp.asarray(a), jnp.asarray(np.ascontiguousarray(a.T))))
    groups = jnp.stack(cols, axis=-1)
    return groups.sum(axis=-1), groups
